# Initial kernel scaffold; baseline (speedup 1.0000x reference)
import jax
import jax.numpy as jnp
from jax.experimental import pallas as pl


def kernel(sparse_fea, stk_coor, n_stk_center, W, bconv, gamma, beta):
    raise NotImplementedError("write your pallas kernel here")


# speedup vs baseline: 2.5292x; 2.5292x over previous
import functools

import jax
import jax.numpy as jnp
from jax import lax
from jax.experimental import pallas as pl
from jax.experimental.pallas import tpu as pltpu
from jax.experimental.pallas import tpu_sc as plsc

SP_NEAR_K = 8
M_CENTER = 1024
F32 = jnp.float32
I32 = jnp.int32


def _xla_sum_c(sq3):
    acc = sq3[0]
    for k in range(1, 16):
        acc = acc + sq3[k]
    t = acc[0:4] + acc[4:8]
    t = t[0:2] + t[2:4]
    return t[0:1] + t[1:2]


def _fps_body(x3_ref, xn_ref, idx_ref):
    x3 = x3_ref[0]
    n = x3.shape[-1]
    lane_n = lax.broadcasted_iota(I32, (1, n), 1)
    lane_m = lax.broadcasted_iota(I32, (1, M_CENTER), 1)

    def body(i, state):
        dist, far, idxs = state
        idxs = jnp.where(lane_m == i, far, idxs)
        row = xn_ref[0, pl.ds(far, 1), :]
        cen = jnp.transpose(row).reshape(16, 8, 1)
        diff = x3 - cen
        d = _xla_sum_c(diff * diff)
        dist = jnp.minimum(dist, d)
        mx = jnp.max(dist)
        cand = jnp.where(dist == mx, lane_n, n)
        far = jnp.min(cand)
        return dist, far, idxs

    init = (jnp.full((1, n), 1e10, F32), jnp.zeros((), I32),
            jnp.zeros((1, M_CENTER), I32))
    _, _, idxs = lax.fori_loop(0, M_CENTER, body, init)
    idx_ref[0] = idxs


def _fps_call(x3, xn, interpret=False):
    b = x3.shape[0]
    return pl.pallas_call(
        _fps_body,
        grid=(b,),
        in_specs=[
            pl.BlockSpec((1, 16, 8, x3.shape[-1]), lambda i: (i, 0, 0, 0)),
            pl.BlockSpec((1, xn.shape[1], xn.shape[2]), lambda i: (i, 0, 0)),
        ],
        out_specs=pl.BlockSpec((1, 1, M_CENTER), lambda i: (i, 0, 0)),
        out_shape=jax.ShapeDtypeStruct((b, 1, M_CENTER), I32),
        interpret=interpret,
    )(x3, xn)


def _sc_gather(table, flat_idx, interpret=False):
    v, d = table.shape
    bsz = flat_idx.shape[0]
    nw = 32
    rows_per_w = bsz // nw
    chunks = rows_per_w // 128
    mesh = plsc.VectorSubcoreMesh(core_axis_name="c", subcore_axis_name="s")

    @functools.partial(
        pl.kernel, mesh=mesh, interpret=interpret,
        out_type=jax.ShapeDtypeStruct((bsz, d), F32),
        scratch_types=[
            pltpu.VMEM((128,), I32),
            pltpu.VMEM((128, d), F32),
            pltpu.SemaphoreType.DMA,
        ],
    )
    def k(table_hbm, idx_hbm, out_hbm, idx_v, rows_v, sem):
        wid = lax.axis_index("s") * 2 + lax.axis_index("c")
        for ch in range(chunks):
            base = wid * rows_per_w + ch * 128
            pltpu.sync_copy(idx_hbm.at[pl.ds(base, 128)], idx_v)
            pltpu.async_copy(table_hbm.at[idx_v], rows_v, sem).wait()
            pltpu.sync_copy(rows_v, out_hbm.at[pl.ds(base, 128)])

    return k(table, flat_idx)


def _knn_mix_body(x3_ref, x3s_ref, xsel_ref, xall_ref, fea_ref, w_ref,
                  knn_ref, z1_ref, zc_ref):
    n = x3_ref.shape[-1]
    m = x3s_ref.shape[-1]

    x3 = x3_ref[0]
    x3s = x3s_ref[0]
    sq_all = _xla_sum_c(x3 * x3)
    sq_sel = _xla_sum_c(x3s * x3s)
    sq_sel_col = jnp.transpose(sq_sel)

    a16 = xsel_ref[0].astype(jnp.bfloat16)
    b16 = xall_ref[0].astype(jnp.bfloat16)
    dot = lax.dot_general(a16, b16, (((1,), (1,)), ((), ())),
                          preferred_element_type=F32)
    d = (sq_sel_col + sq_all) - 2.0 * dot

    iota_n = lax.broadcasted_iota(I32, (m, n), 1)
    cols = []
    for _ in range(SP_NEAR_K):
        mn = jnp.min(d, axis=1, keepdims=True)
        cand = jnp.where(d == mn, iota_n, n)
        idx_t = jnp.min(cand, axis=1, keepdims=True)
        d = jnp.where(iota_n == idx_t, jnp.inf, d)
        cols.append(idx_t)
    knn_ref[0] = jnp.concatenate(cols, axis=1)

    fea16 = fea_ref[0].astype(jnp.bfloat16)
    w = w_ref[...]
    c = w.shape[1] // 2
    w1_16 = w[:, 0:c].astype(jnp.bfloat16)
    wc_16 = (w[:, c:2 * c] - w[:, 0:c]).astype(jnp.bfloat16)
    z1_ref[0] = lax.dot_general(fea16, w1_16, (((1,), (1,)), ((), ())),
                                preferred_element_type=F32)
    zc_ref[0] = lax.dot_general(fea16, wc_16, (((1,), (1,)), ((), ())),
                                preferred_element_type=F32)


def _knn_mix_call(x3, x3s, xsel, xall, feaT, w, interpret=False):
    b, _, _, n = x3.shape
    m = x3s.shape[-1]
    co = w.shape[0]
    return pl.pallas_call(
        _knn_mix_body,
        grid=(b,),
        in_specs=[
            pl.BlockSpec((1, 16, 8, n), lambda i: (i, 0, 0, 0)),
            pl.BlockSpec((1, 16, 8, m), lambda i: (i, 0, 0, 0)),
            pl.BlockSpec((1, m, 128), lambda i: (i, 0, 0)),
            pl.BlockSpec((1, n, 128), lambda i: (i, 0, 0)),
            pl.BlockSpec((1, n, 256), lambda i: (i, 0, 0)),
            pl.BlockSpec((co, w.shape[1]), lambda i: (0, 0)),
        ],
        out_specs=[
            pl.BlockSpec((1, m, SP_NEAR_K), lambda i: (i, 0, 0)),
            pl.BlockSpec((1, n, co), lambda i: (i, 0, 0)),
            pl.BlockSpec((1, n, co), lambda i: (i, 0, 0)),
        ],
        out_shape=[
            jax.ShapeDtypeStruct((b, m, SP_NEAR_K), I32),
            jax.ShapeDtypeStruct((b, n, co), F32),
            jax.ShapeDtypeStruct((b, n, co), F32),
        ],
        interpret=interpret,
    )(x3, x3s, xsel, xall, feaT, w)


def _pool_stats_body(g_ref, c_ref, bconv_ref, part_ref, mx_ref, mn_ref):
    g = g_ref[0]
    cb = c_ref[0] + bconv_ref[...]
    y0 = g[0] + cb
    mx = y0
    mn = y0
    s = y0
    ss = y0 * y0
    for k in range(1, SP_NEAR_K):
        yk = g[k] + cb
        mx = jnp.maximum(mx, yk)
        mn = jnp.minimum(mn, yk)
        s = s + yk
        ss = ss + yk * yk
    mx_ref[0] = mx
    mn_ref[0] = mn
    part_ref[0] = jnp.concatenate(
        [jnp.sum(s, axis=0, keepdims=True),
         jnp.sum(ss, axis=0, keepdims=True)], axis=0)


def _pool_stats_call(g4, c4, bconv2, interpret=False):
    b, _, m, co = g4.shape
    return pl.pallas_call(
        _pool_stats_body,
        grid=(b,),
        in_specs=[
            pl.BlockSpec((1, SP_NEAR_K, m, co), lambda i: (i, 0, 0, 0)),
            pl.BlockSpec((1, m, co), lambda i: (i, 0, 0)),
            pl.BlockSpec((1, co), lambda i: (0, 0)),
        ],
        out_specs=[
            pl.BlockSpec((1, 2, co), lambda i: (i, 0, 0)),
            pl.BlockSpec((1, m, co), lambda i: (i, 0, 0)),
            pl.BlockSpec((1, m, co), lambda i: (i, 0, 0)),
        ],
        out_shape=[
            jax.ShapeDtypeStruct((b, 2, co), F32),
            jax.ShapeDtypeStruct((b, m, co), F32),
            jax.ShapeDtypeStruct((b, m, co), F32),
        ],
        interpret=interpret,
    )(g4, c4, bconv2)


def _finalize_body(part_ref, mx_ref, mn_ref, gamma_ref, beta_ref, out_ref,
                   *, count):
    p = part_ref[...]
    s2 = jnp.sum(p[:, 0, :], axis=0, keepdims=True)
    ss2 = jnp.sum(p[:, 1, :], axis=0, keepdims=True)
    mu = s2 / count
    var = ss2 / count - mu * mu
    denom = jnp.sqrt(var + 1e-5)
    gamma = gamma_ref[...]
    beta = beta_ref[...]
    chosen = jnp.where(gamma >= 0.0, mx_ref[0], mn_ref[0])
    t = (chosen - mu) / denom * gamma + beta
    out_ref[0] = jnp.transpose(jax.nn.relu(t))


def _finalize_call(part, mx, mn, gamma2, beta2, interpret=False):
    b, m, co = mx.shape
    count = float(b * m * SP_NEAR_K)
    return pl.pallas_call(
        functools.partial(_finalize_body, count=count),
        grid=(b,),
        in_specs=[
            pl.BlockSpec((b, 2, co), lambda i: (0, 0, 0)),
            pl.BlockSpec((1, m, co), lambda i: (i, 0, 0)),
            pl.BlockSpec((1, m, co), lambda i: (i, 0, 0)),
            pl.BlockSpec((1, co), lambda i: (0, 0)),
            pl.BlockSpec((1, co), lambda i: (0, 0)),
        ],
        out_specs=pl.BlockSpec((1, co, m), lambda i: (i, 0, 0)),
        out_shape=jax.ShapeDtypeStruct((b, co, m), F32),
        interpret=interpret,
    )(part, mx, mn, gamma2, beta2)


def kernel(sparse_fea, stk_coor, n_stk_center, W, bconv, gamma, beta):
    b, n, cc = stk_coor.shape
    co = W.shape[0]
    m = M_CENTER

    x3 = stk_coor.reshape(b, n, 16, 8).transpose(0, 2, 3, 1)

    fps_idx = _fps_call(x3, stk_coor).reshape(b, m)
    fps_idx = fps_idx + (jnp.asarray(n_stk_center, I32) - jnp.asarray(m, I32))

    boff = (jnp.arange(b, dtype=I32) * n)[:, None]
    fps_flat = (fps_idx + boff).reshape(b * m)

    stk_sampled = _sc_gather(stk_coor.reshape(b * n, cc), fps_flat)
    stk_sampled = stk_sampled.reshape(b, m, cc)

    x3s = stk_sampled.reshape(b, m, 16, 8).transpose(0, 2, 3, 1)
    feaT = sparse_fea.transpose(0, 2, 1)

    knn_idx, z1t, zct = _knn_mix_call(x3, x3s, stk_sampled, stk_coor, feaT, W)

    knn_km = knn_idx.transpose(0, 2, 1)
    knn_flat = (knn_km + boff[:, :, None]).reshape(b * SP_NEAR_K * m)

    g = _sc_gather(z1t.reshape(b * n, co), knn_flat)
    c = _sc_gather(zct.reshape(b * n, co), fps_flat)
    g4 = g.reshape(b, SP_NEAR_K, m, co)
    c4 = c.reshape(b, m, co)

    part, mx, mn = _pool_stats_call(g4, c4, bconv.reshape(1, co))
    out_fea = _finalize_call(part, mx, mn, gamma.reshape(1, co),
                             beta.reshape(1, co))
    return (out_fea, stk_sampled)


# speedup vs baseline: 3.7433x; 1.4800x over previous
import functools

import jax
import jax.numpy as jnp
from jax import lax
from jax.experimental import pallas as pl
from jax.experimental.pallas import tpu as pltpu
from jax.experimental.pallas import tpu_sc as plsc

SP_NEAR_K = 8
M_CENTER = 1024
F32 = jnp.float32
I32 = jnp.int32


def _xla_sum_c(sq3):
    acc = sq3[0]
    for k in range(1, 16):
        acc = acc + sq3[k]
    t = acc[0:4] + acc[4:8]
    t = t[0:2] + t[2:4]
    return t[0:1] + t[1:2]


def _fps_body(x3b_ref, xn_ref, idx_ref, *, b, n):
    x3b = x3b_ref[...]
    lane_n = lax.broadcasted_iota(I32, (1, n), 1)
    lane_m = lax.broadcasted_iota(I32, (1, M_CENTER), 1)

    def body(i, state):
        dists, fars, idxss = state
        idxss = tuple(jnp.where(lane_m == i, fars[bb], idxss[bb])
                      for bb in range(b))
        cens = []
        for bb in range(b):
            row = xn_ref[pl.ds(fars[bb] + bb * n, 1), :]
            cens.append(jnp.transpose(row).reshape(16, 8, 1))
        cen = jnp.concatenate(cens, axis=1)
        diff = x3b - cen
        sq = diff * diff
        acc = sq[0]
        for k in range(1, 16):
            acc = acc + sq[k]
        a3 = acc.reshape(b, 8, n)
        t = a3[:, 0:4] + a3[:, 4:8]
        t = t[:, 0:2] + t[:, 2:4]
        d = t[:, 0:1] + t[:, 1:2]
        new_d, new_f = [], []
        for bb in range(b):
            dist = jnp.minimum(dists[bb], d[bb])
            mx = jnp.max(dist)
            cand = jnp.where(dist == mx, lane_n, n)
            new_f.append(jnp.min(cand))
            new_d.append(dist)
        return tuple(new_d), tuple(new_f), idxss

    init = (tuple(jnp.full((1, n), 1e10, F32) for _ in range(b)),
            tuple(jnp.zeros((), I32) for _ in range(b)),
            tuple(jnp.zeros((1, M_CENTER), I32) for _ in range(b)))
    _, _, idxss = lax.fori_loop(0, M_CENTER, body, init)
    for bb in range(b):
        idx_ref[bb] = idxss[bb]


def _fps_call(x3b, xn_flat, interpret=False):
    b = xn_flat.shape[0] // x3b.shape[-1]
    n = x3b.shape[-1]
    return pl.pallas_call(
        functools.partial(_fps_body, b=b, n=n),
        in_specs=[
            pl.BlockSpec(x3b.shape, lambda: (0, 0, 0)),
            pl.BlockSpec(xn_flat.shape, lambda: (0, 0)),
        ],
        out_specs=pl.BlockSpec((b, 1, M_CENTER), lambda: (0, 0, 0)),
        out_shape=jax.ShapeDtypeStruct((b, 1, M_CENTER), I32),
        interpret=interpret,
    )(x3b, xn_flat)


def _sc_gather(table, flat_idx, interpret=False):
    v, d = table.shape
    bsz = flat_idx.shape[0]
    nw = 32
    rows_per_w = bsz // nw
    chunks = rows_per_w // 128
    mesh = plsc.VectorSubcoreMesh(core_axis_name="c", subcore_axis_name="s")

    @functools.partial(
        pl.kernel, mesh=mesh, interpret=interpret,
        out_type=jax.ShapeDtypeStruct((bsz, d), F32),
        scratch_types=[
            pltpu.VMEM((128,), I32),
            pltpu.VMEM((128, d), F32),
            pltpu.SemaphoreType.DMA,
        ],
    )
    def k(table_hbm, idx_hbm, out_hbm, idx_v, rows_v, sem):
        wid = lax.axis_index("s") * 2 + lax.axis_index("c")
        for ch in range(chunks):
            base = wid * rows_per_w + ch * 128
            pltpu.sync_copy(idx_hbm.at[pl.ds(base, 128)], idx_v)
            pltpu.async_copy(table_hbm.at[idx_v], rows_v, sem).wait()
            pltpu.sync_copy(rows_v, out_hbm.at[pl.ds(base, 128)])

    return k(table, flat_idx)


def _knn_mix_body(x3_ref, x3s_ref, xsel_ref, xall_ref, fea_ref, w_ref,
                  knn_ref, z1_ref, zc_ref):
    n = x3_ref.shape[-1]
    m = x3s_ref.shape[-1]

    x3 = x3_ref[0]
    x3s = x3s_ref[0]
    sq_all = _xla_sum_c(x3 * x3)
    sq_sel = _xla_sum_c(x3s * x3s)
    sq_sel_col = jnp.transpose(sq_sel)

    a16 = xsel_ref[0].astype(jnp.bfloat16)
    b16 = xall_ref[0].astype(jnp.bfloat16)
    dot = lax.dot_general(a16, b16, (((1,), (1,)), ((), ())),
                          preferred_element_type=F32)
    d = (sq_sel_col + sq_all) - 2.0 * dot

    iota_n = lax.broadcasted_iota(I32, (m, n), 1)
    cols = []
    for _ in range(SP_NEAR_K):
        mn = jnp.min(d, axis=1, keepdims=True)
        cand = jnp.where(d == mn, iota_n, n)
        idx_t = jnp.min(cand, axis=1, keepdims=True)
        d = jnp.where(iota_n == idx_t, jnp.inf, d)
        cols.append(idx_t)
    knn_ref[0] = jnp.concatenate(cols, axis=1)

    fea16 = fea_ref[0].astype(jnp.bfloat16)
    w = w_ref[...]
    c = w.shape[1] // 2
    w1_16 = w[:, 0:c].astype(jnp.bfloat16)
    wc_16 = (w[:, c:2 * c] - w[:, 0:c]).astype(jnp.bfloat16)
    z1_ref[0] = lax.dot_general(fea16, w1_16, (((1,), (1,)), ((), ())),
                                preferred_element_type=F32)
    zc_ref[0] = lax.dot_general(fea16, wc_16, (((1,), (1,)), ((), ())),
                                preferred_element_type=F32)


def _knn_mix_call(x3, x3s, xsel, xall, feaT, w, interpret=False):
    b, _, _, n = x3.shape
    m = x3s.shape[-1]
    co = w.shape[0]
    return pl.pallas_call(
        _knn_mix_body,
        grid=(b,),
        in_specs=[
            pl.BlockSpec((1, 16, 8, n), lambda i: (i, 0, 0, 0)),
            pl.BlockSpec((1, 16, 8, m), lambda i: (i, 0, 0, 0)),
            pl.BlockSpec((1, m, 128), lambda i: (i, 0, 0)),
            pl.BlockSpec((1, n, 128), lambda i: (i, 0, 0)),
            pl.BlockSpec((1, n, 256), lambda i: (i, 0, 0)),
            pl.BlockSpec((co, w.shape[1]), lambda i: (0, 0)),
        ],
        out_specs=[
            pl.BlockSpec((1, m, SP_NEAR_K), lambda i: (i, 0, 0)),
            pl.BlockSpec((1, n, co), lambda i: (i, 0, 0)),
            pl.BlockSpec((1, n, co), lambda i: (i, 0, 0)),
        ],
        out_shape=[
            jax.ShapeDtypeStruct((b, m, SP_NEAR_K), I32),
            jax.ShapeDtypeStruct((b, n, co), F32),
            jax.ShapeDtypeStruct((b, n, co), F32),
        ],
        interpret=interpret,
    )(x3, x3s, xsel, xall, feaT, w)


def _pool_stats_body(g_ref, c_ref, bconv_ref, part_ref, mx_ref, mn_ref):
    g = g_ref[0]
    cb = c_ref[0] + bconv_ref[...]
    y0 = g[0] + cb
    mx = y0
    mn = y0
    s = y0
    ss = y0 * y0
    for k in range(1, SP_NEAR_K):
        yk = g[k] + cb
        mx = jnp.maximum(mx, yk)
        mn = jnp.minimum(mn, yk)
        s = s + yk
        ss = ss + yk * yk
    mx_ref[0] = mx
    mn_ref[0] = mn
    part_ref[0] = jnp.concatenate(
        [jnp.sum(s, axis=0, keepdims=True),
         jnp.sum(ss, axis=0, keepdims=True)], axis=0)


def _pool_stats_call(g4, c4, bconv2, interpret=False):
    b, _, m, co = g4.shape
    return pl.pallas_call(
        _pool_stats_body,
        grid=(b,),
        in_specs=[
            pl.BlockSpec((1, SP_NEAR_K, m, co), lambda i: (i, 0, 0, 0)),
            pl.BlockSpec((1, m, co), lambda i: (i, 0, 0)),
            pl.BlockSpec((1, co), lambda i: (0, 0)),
        ],
        out_specs=[
            pl.BlockSpec((1, 2, co), lambda i: (i, 0, 0)),
            pl.BlockSpec((1, m, co), lambda i: (i, 0, 0)),
            pl.BlockSpec((1, m, co), lambda i: (i, 0, 0)),
        ],
        out_shape=[
            jax.ShapeDtypeStruct((b, 2, co), F32),
            jax.ShapeDtypeStruct((b, m, co), F32),
            jax.ShapeDtypeStruct((b, m, co), F32),
        ],
        interpret=interpret,
    )(g4, c4, bconv2)


def _finalize_body(part_ref, mx_ref, mn_ref, gamma_ref, beta_ref, out_ref,
                   *, count):
    p = part_ref[...]
    s2 = jnp.sum(p[:, 0, :], axis=0, keepdims=True)
    ss2 = jnp.sum(p[:, 1, :], axis=0, keepdims=True)
    mu = s2 / count
    var = ss2 / count - mu * mu
    denom = jnp.sqrt(var + 1e-5)
    gamma = gamma_ref[...]
    beta = beta_ref[...]
    chosen = jnp.where(gamma >= 0.0, mx_ref[0], mn_ref[0])
    t = (chosen - mu) / denom * gamma + beta
    out_ref[0] = jnp.transpose(jax.nn.relu(t))


def _finalize_call(part, mx, mn, gamma2, beta2, interpret=False):
    b, m, co = mx.shape
    count = float(b * m * SP_NEAR_K)
    return pl.pallas_call(
        functools.partial(_finalize_body, count=count),
        grid=(b,),
        in_specs=[
            pl.BlockSpec((b, 2, co), lambda i: (0, 0, 0)),
            pl.BlockSpec((1, m, co), lambda i: (i, 0, 0)),
            pl.BlockSpec((1, m, co), lambda i: (i, 0, 0)),
            pl.BlockSpec((1, co), lambda i: (0, 0)),
            pl.BlockSpec((1, co), lambda i: (0, 0)),
        ],
        out_specs=pl.BlockSpec((1, co, m), lambda i: (i, 0, 0)),
        out_shape=jax.ShapeDtypeStruct((b, co, m), F32),
        interpret=interpret,
    )(part, mx, mn, gamma2, beta2)


def kernel(sparse_fea, stk_coor, n_stk_center, W, bconv, gamma, beta):
    b, n, cc = stk_coor.shape
    co = W.shape[0]
    m = M_CENTER

    x3 = stk_coor.reshape(b, n, 16, 8).transpose(0, 2, 3, 1)
    x3b = stk_coor.reshape(b, n, 16, 8).transpose(2, 0, 3, 1).reshape(16, 8 * b, n)

    fps_idx = _fps_call(x3b, stk_coor.reshape(b * n, cc)).reshape(b, m)
    fps_idx = fps_idx + (jnp.asarray(n_stk_center, I32) - jnp.asarray(m, I32))

    boff = (jnp.arange(b, dtype=I32) * n)[:, None]
    fps_flat = (fps_idx + boff).reshape(b * m)

    stk_sampled = _sc_gather(stk_coor.reshape(b * n, cc), fps_flat)
    stk_sampled = stk_sampled.reshape(b, m, cc)

    x3s = stk_sampled.reshape(b, m, 16, 8).transpose(0, 2, 3, 1)
    feaT = sparse_fea.transpose(0, 2, 1)

    knn_idx, z1t, zct = _knn_mix_call(x3, x3s, stk_sampled, stk_coor, feaT, W)

    knn_km = knn_idx.transpose(0, 2, 1)
    knn_flat = (knn_km + boff[:, :, None]).reshape(b * SP_NEAR_K * m)

    g = _sc_gather(z1t.reshape(b * n, co), knn_flat)
    c = _sc_gather(zct.reshape(b * n, co), fps_flat)
    g4 = g.reshape(b, SP_NEAR_K, m, co)
    c4 = c.reshape(b, m, co)

    part, mx, mn = _pool_stats_call(g4, c4, bconv.reshape(1, co))
    out_fea = _finalize_call(part, mx, mn, gamma.reshape(1, co),
                             beta.reshape(1, co))
    return (out_fea, stk_sampled)


# speedup vs baseline: 4.0998x; 1.0953x over previous
import functools

import jax
import jax.numpy as jnp
from jax import lax
from jax.experimental import pallas as pl
from jax.experimental.pallas import tpu as pltpu
from jax.experimental.pallas import tpu_sc as plsc

SP_NEAR_K = 8
M_CENTER = 1024
F32 = jnp.float32
I32 = jnp.int32


def _xla_sum_c(sq3):
    acc = sq3[0]
    for k in range(1, 16):
        acc = acc + sq3[k]
    t = acc[0:4] + acc[4:8]
    t = t[0:2] + t[2:4]
    return t[0:1] + t[1:2]


def _fps_body(x3b_ref, xn_ref, idx_ref, *, b, n):
    lane_n = lax.broadcasted_iota(I32, (1, n), 1)
    lane_m = lax.broadcasted_iota(I32, (1, M_CENTER), 1)

    def body(i, state):
        dists, fars, idxss = state
        idxss = tuple(jnp.where(lane_m == i, fars[bb], idxss[bb])
                      for bb in range(b))
        cens = [None] * b
        for bb in range(b):
            row = xn_ref[pl.ds(fars[bb] + bb * n, 1), :]
            cens[bb] = jnp.transpose(row).reshape(16, 8, 1)
        new_d, new_f = [], []
        ck = 512
        for bb in range(b):
            parts = []
            for c0 in range(0, n, ck):
                xc = x3b_ref[:, bb * 8:(bb + 1) * 8, c0:c0 + ck]
                diff = xc - cens[bb]
                sq = diff * diff
                acc = sq[0]
                for k in range(1, 16):
                    acc = acc + sq[k]
                t = acc[0:4] + acc[4:8]
                t = t[0:2] + t[2:4]
                parts.append(t[0:1] + t[1:2])
            d = jnp.concatenate(parts, axis=1)
            dist = jnp.minimum(dists[bb], d)
            mx = jnp.max(dist)
            cand = jnp.where(dist == mx, lane_n, n)
            new_f.append(jnp.min(cand))
            new_d.append(dist)
        return tuple(new_d), tuple(new_f), idxss

    init = (tuple(jnp.full((1, n), 1e10, F32) for _ in range(b)),
            tuple(jnp.zeros((), I32) for _ in range(b)),
            tuple(jnp.zeros((1, M_CENTER), I32) for _ in range(b)))
    _, _, idxss = lax.fori_loop(0, M_CENTER, body, init)
    for bb in range(b):
        idx_ref[bb] = idxss[bb]


def _fps_call(x3b, xn_flat, interpret=False):
    b = xn_flat.shape[0] // x3b.shape[-1]
    n = x3b.shape[-1]
    return pl.pallas_call(
        functools.partial(_fps_body, b=b, n=n),
        in_specs=[
            pl.BlockSpec(x3b.shape, lambda: (0, 0, 0)),
            pl.BlockSpec(xn_flat.shape, lambda: (0, 0)),
        ],
        out_specs=pl.BlockSpec((b, 1, M_CENTER), lambda: (0, 0, 0)),
        out_shape=jax.ShapeDtypeStruct((b, 1, M_CENTER), I32),
        interpret=interpret,
    )(x3b, xn_flat)


def _sc_gather(table, flat_idx, interpret=False):
    v, d = table.shape
    bsz = flat_idx.shape[0]
    nw = 32
    rows_per_w = bsz // nw
    chunks = rows_per_w // 128
    mesh = plsc.VectorSubcoreMesh(core_axis_name="c", subcore_axis_name="s")

    @functools.partial(
        pl.kernel, mesh=mesh, interpret=interpret,
        out_type=jax.ShapeDtypeStruct((bsz, d), F32),
        scratch_types=[
            pltpu.VMEM((128,), I32),
            pltpu.VMEM((128, d), F32),
            pltpu.SemaphoreType.DMA,
        ],
    )
    def k(table_hbm, idx_hbm, out_hbm, idx_v, rows_v, sem):
        wid = lax.axis_index("s") * 2 + lax.axis_index("c")
        for ch in range(chunks):
            base = wid * rows_per_w + ch * 128
            pltpu.sync_copy(idx_hbm.at[pl.ds(base, 128)], idx_v)
            pltpu.async_copy(table_hbm.at[idx_v], rows_v, sem).wait()
            pltpu.sync_copy(rows_v, out_hbm.at[pl.ds(base, 128)])

    return k(table, flat_idx)


def _knn_mix_body(x3_ref, x3s_ref, xsel_ref, xall_ref, fea_ref, w_ref,
                  knn_ref, z1_ref, zc_ref):
    n = x3_ref.shape[-1]
    m = x3s_ref.shape[-1]

    x3 = x3_ref[0]
    x3s = x3s_ref[0]
    sq_all = _xla_sum_c(x3 * x3)
    sq_sel = _xla_sum_c(x3s * x3s)
    sq_sel_col = jnp.transpose(sq_sel)

    a16 = xsel_ref[0].astype(jnp.bfloat16)
    b16 = xall_ref[0].astype(jnp.bfloat16)
    dot = lax.dot_general(a16, b16, (((1,), (1,)), ((), ())),
                          preferred_element_type=F32)
    d = (sq_sel_col + sq_all) - 2.0 * dot

    iota_n = lax.broadcasted_iota(I32, (m, n), 1)
    cols = []
    for _ in range(SP_NEAR_K):
        mn = jnp.min(d, axis=1, keepdims=True)
        cand = jnp.where(d == mn, iota_n, n)
        idx_t = jnp.min(cand, axis=1, keepdims=True)
        d = jnp.where(iota_n == idx_t, jnp.inf, d)
        cols.append(idx_t)
    knn_ref[0] = jnp.concatenate(cols, axis=1)

    fea16 = fea_ref[0].astype(jnp.bfloat16)
    w = w_ref[...]
    c = w.shape[1] // 2
    w1_16 = w[:, 0:c].astype(jnp.bfloat16)
    wc_16 = (w[:, c:2 * c] - w[:, 0:c]).astype(jnp.bfloat16)
    z1_ref[0] = lax.dot_general(fea16, w1_16, (((1,), (1,)), ((), ())),
                                preferred_element_type=F32)
    zc_ref[0] = lax.dot_general(fea16, wc_16, (((1,), (1,)), ((), ())),
                                preferred_element_type=F32)


def _knn_mix_call(x3, x3s, xsel, xall, feaT, w, interpret=False):
    b, _, _, n = x3.shape
    m = x3s.shape[-1]
    co = w.shape[0]
    return pl.pallas_call(
        _knn_mix_body,
        grid=(b,),
        in_specs=[
            pl.BlockSpec((1, 16, 8, n), lambda i: (i, 0, 0, 0)),
            pl.BlockSpec((1, 16, 8, m), lambda i: (i, 0, 0, 0)),
            pl.BlockSpec((1, m, 128), lambda i: (i, 0, 0)),
            pl.BlockSpec((1, n, 128), lambda i: (i, 0, 0)),
            pl.BlockSpec((1, n, 256), lambda i: (i, 0, 0)),
            pl.BlockSpec((co, w.shape[1]), lambda i: (0, 0)),
        ],
        out_specs=[
            pl.BlockSpec((1, m, SP_NEAR_K), lambda i: (i, 0, 0)),
            pl.BlockSpec((1, n, co), lambda i: (i, 0, 0)),
            pl.BlockSpec((1, n, co), lambda i: (i, 0, 0)),
        ],
        out_shape=[
            jax.ShapeDtypeStruct((b, m, SP_NEAR_K), I32),
            jax.ShapeDtypeStruct((b, n, co), F32),
            jax.ShapeDtypeStruct((b, n, co), F32),
        ],
        interpret=interpret,
    )(x3, x3s, xsel, xall, feaT, w)


def _pool_stats_body(g_ref, c_ref, bconv_ref, part_ref, mx_ref, mn_ref):
    g = g_ref[0]
    cb = c_ref[0] + bconv_ref[...]
    y0 = g[0] + cb
    mx = y0
    mn = y0
    s = y0
    ss = y0 * y0
    for k in range(1, SP_NEAR_K):
        yk = g[k] + cb
        mx = jnp.maximum(mx, yk)
        mn = jnp.minimum(mn, yk)
        s = s + yk
        ss = ss + yk * yk
    mx_ref[0] = mx
    mn_ref[0] = mn
    part_ref[0] = jnp.concatenate(
        [jnp.sum(s, axis=0, keepdims=True),
         jnp.sum(ss, axis=0, keepdims=True)], axis=0)


def _pool_stats_call(g4, c4, bconv2, interpret=False):
    b, _, m, co = g4.shape
    return pl.pallas_call(
        _pool_stats_body,
        grid=(b,),
        in_specs=[
            pl.BlockSpec((1, SP_NEAR_K, m, co), lambda i: (i, 0, 0, 0)),
            pl.BlockSpec((1, m, co), lambda i: (i, 0, 0)),
            pl.BlockSpec((1, co), lambda i: (0, 0)),
        ],
        out_specs=[
            pl.BlockSpec((1, 2, co), lambda i: (i, 0, 0)),
            pl.BlockSpec((1, m, co), lambda i: (i, 0, 0)),
            pl.BlockSpec((1, m, co), lambda i: (i, 0, 0)),
        ],
        out_shape=[
            jax.ShapeDtypeStruct((b, 2, co), F32),
            jax.ShapeDtypeStruct((b, m, co), F32),
            jax.ShapeDtypeStruct((b, m, co), F32),
        ],
        interpret=interpret,
    )(g4, c4, bconv2)


def _finalize_body(part_ref, mx_ref, mn_ref, gamma_ref, beta_ref, out_ref,
                   *, count):
    p = part_ref[...]
    s2 = jnp.sum(p[:, 0, :], axis=0, keepdims=True)
    ss2 = jnp.sum(p[:, 1, :], axis=0, keepdims=True)
    mu = s2 / count
    var = ss2 / count - mu * mu
    denom = jnp.sqrt(var + 1e-5)
    gamma = gamma_ref[...]
    beta = beta_ref[...]
    chosen = jnp.where(gamma >= 0.0, mx_ref[0], mn_ref[0])
    t = (chosen - mu) / denom * gamma + beta
    out_ref[0] = jnp.transpose(jax.nn.relu(t))


def _finalize_call(part, mx, mn, gamma2, beta2, interpret=False):
    b, m, co = mx.shape
    count = float(b * m * SP_NEAR_K)
    return pl.pallas_call(
        functools.partial(_finalize_body, count=count),
        grid=(b,),
        in_specs=[
            pl.BlockSpec((b, 2, co), lambda i: (0, 0, 0)),
            pl.BlockSpec((1, m, co), lambda i: (i, 0, 0)),
            pl.BlockSpec((1, m, co), lambda i: (i, 0, 0)),
            pl.BlockSpec((1, co), lambda i: (0, 0)),
            pl.BlockSpec((1, co), lambda i: (0, 0)),
        ],
        out_specs=pl.BlockSpec((1, co, m), lambda i: (i, 0, 0)),
        out_shape=jax.ShapeDtypeStruct((b, co, m), F32),
        interpret=interpret,
    )(part, mx, mn, gamma2, beta2)


def kernel(sparse_fea, stk_coor, n_stk_center, W, bconv, gamma, beta):
    b, n, cc = stk_coor.shape
    co = W.shape[0]
    m = M_CENTER

    x3 = stk_coor.reshape(b, n, 16, 8).transpose(0, 2, 3, 1)
    x3b = stk_coor.reshape(b, n, 16, 8).transpose(2, 0, 3, 1).reshape(16, 8 * b, n)

    fps_idx = _fps_call(x3b, stk_coor.reshape(b * n, cc)).reshape(b, m)
    fps_idx = fps_idx + (jnp.asarray(n_stk_center, I32) - jnp.asarray(m, I32))

    boff = (jnp.arange(b, dtype=I32) * n)[:, None]
    fps_flat = (fps_idx + boff).reshape(b * m)

    stk_sampled = _sc_gather(stk_coor.reshape(b * n, cc), fps_flat)
    stk_sampled = stk_sampled.reshape(b, m, cc)

    x3s = stk_sampled.reshape(b, m, 16, 8).transpose(0, 2, 3, 1)
    feaT = sparse_fea.transpose(0, 2, 1)

    knn_idx, z1t, zct = _knn_mix_call(x3, x3s, stk_sampled, stk_coor, feaT, W)

    knn_km = knn_idx.transpose(0, 2, 1)
    knn_flat = (knn_km + boff[:, :, None]).reshape(b * SP_NEAR_K * m)

    g = _sc_gather(z1t.reshape(b * n, co), knn_flat)
    c = _sc_gather(zct.reshape(b * n, co), fps_flat)
    g4 = g.reshape(b, SP_NEAR_K, m, co)
    c4 = c.reshape(b, m, co)

    part, mx, mn = _pool_stats_call(g4, c4, bconv.reshape(1, co))
    out_fea = _finalize_call(part, mx, mn, gamma.reshape(1, co),
                             beta.reshape(1, co))
    return (out_fea, stk_sampled)


# speedup vs baseline: 4.2835x; 1.0448x over previous
import functools

import jax
import jax.numpy as jnp
from jax import lax
from jax.experimental import pallas as pl
from jax.experimental.pallas import tpu as pltpu
from jax.experimental.pallas import tpu_sc as plsc

SP_NEAR_K = 8
M_CENTER = 1024
F32 = jnp.float32
I32 = jnp.int32


def _xla_sum_c(sq3):
    acc = sq3[0]
    for k in range(1, 16):
        acc = acc + sq3[k]
    t = acc[0:4] + acc[4:8]
    t = t[0:2] + t[2:4]
    return t[0:1] + t[1:2]


def _fps_body(x3b_ref, xn_ref, idx_ref, *, b, n):
    lane_n = lax.broadcasted_iota(I32, (1, n), 1)
    lane_m = lax.broadcasted_iota(I32, (1, M_CENTER), 1)

    def body(i, state):
        dists, fars, idxss = state
        idxss = tuple(jnp.where(lane_m == i, fars[bb], idxss[bb])
                      for bb in range(b))
        cens = [None] * b
        for bb in range(b):
            row = xn_ref[pl.ds(fars[bb] + bb * n, 1), :]
            cens[bb] = jnp.transpose(row).reshape(16, 8, 1)
        new_d, new_f = [], []
        for bb in range(b):
            acc = None
            for k in range(16):
                xc = x3b_ref[k, bb * 8:(bb + 1) * 8, :]
                dk = xc - cens[bb][k]
                sqk = dk * dk
                acc = sqk if acc is None else acc + sqk
            t = acc[0:4] + acc[4:8]
            t = t[0:2] + t[2:4]
            d = t[0:1] + t[1:2]
            dist = jnp.minimum(dists[bb], d)
            mx = jnp.max(dist)
            cand = jnp.where(dist == mx, lane_n, n)
            new_f.append(jnp.min(cand))
            new_d.append(dist)
        return tuple(new_d), tuple(new_f), idxss

    init = (tuple(jnp.full((1, n), 1e10, F32) for _ in range(b)),
            tuple(jnp.zeros((), I32) for _ in range(b)),
            tuple(jnp.zeros((1, M_CENTER), I32) for _ in range(b)))
    _, _, idxss = lax.fori_loop(0, M_CENTER, body, init)
    for bb in range(b):
        idx_ref[bb] = idxss[bb]


def _fps_call(x3b, xn_flat, interpret=False):
    b = xn_flat.shape[0] // x3b.shape[-1]
    n = x3b.shape[-1]
    return pl.pallas_call(
        functools.partial(_fps_body, b=b, n=n),
        in_specs=[
            pl.BlockSpec(x3b.shape, lambda: (0, 0, 0)),
            pl.BlockSpec(xn_flat.shape, lambda: (0, 0)),
        ],
        out_specs=pl.BlockSpec((b, 1, M_CENTER), lambda: (0, 0, 0)),
        out_shape=jax.ShapeDtypeStruct((b, 1, M_CENTER), I32),
        interpret=interpret,
    )(x3b, xn_flat)


def _sc_gather(table, flat_idx, interpret=False):
    v, d = table.shape
    bsz = flat_idx.shape[0]
    nw = 32
    rows_per_w = bsz // nw
    chunks = rows_per_w // 128
    mesh = plsc.VectorSubcoreMesh(core_axis_name="c", subcore_axis_name="s")

    @functools.partial(
        pl.kernel, mesh=mesh, interpret=interpret,
        out_type=jax.ShapeDtypeStruct((bsz, d), F32),
        scratch_types=[
            pltpu.VMEM((128,), I32),
            pltpu.VMEM((128, d), F32),
            pltpu.SemaphoreType.DMA,
        ],
    )
    def k(table_hbm, idx_hbm, out_hbm, idx_v, rows_v, sem):
        wid = lax.axis_index("s") * 2 + lax.axis_index("c")
        for ch in range(chunks):
            base = wid * rows_per_w + ch * 128
            pltpu.sync_copy(idx_hbm.at[pl.ds(base, 128)], idx_v)
            pltpu.async_copy(table_hbm.at[idx_v], rows_v, sem).wait()
            pltpu.sync_copy(rows_v, out_hbm.at[pl.ds(base, 128)])

    return k(table, flat_idx)


def _knn_mix_body(x3_ref, x3s_ref, xsel_ref, xall_ref, fea_ref, w_ref,
                  knn_ref, z1_ref, zc_ref):
    n = x3_ref.shape[-1]
    m = x3s_ref.shape[-1]

    x3 = x3_ref[0]
    x3s = x3s_ref[0]
    sq_all = _xla_sum_c(x3 * x3)
    sq_sel = _xla_sum_c(x3s * x3s)
    sq_sel_col = jnp.transpose(sq_sel)

    a16 = xsel_ref[0].astype(jnp.bfloat16)
    b16 = xall_ref[0].astype(jnp.bfloat16)
    dot = lax.dot_general(a16, b16, (((1,), (1,)), ((), ())),
                          preferred_element_type=F32)
    d = (sq_sel_col + sq_all) - 2.0 * dot

    iota_n = lax.broadcasted_iota(I32, (m, n), 1)
    cols = []
    for _ in range(SP_NEAR_K):
        mn = jnp.min(d, axis=1, keepdims=True)
        cand = jnp.where(d == mn, iota_n, n)
        idx_t = jnp.min(cand, axis=1, keepdims=True)
        d = jnp.where(iota_n == idx_t, jnp.inf, d)
        cols.append(idx_t)
    knn_ref[0] = jnp.concatenate(cols, axis=1)

    fea16 = fea_ref[0].astype(jnp.bfloat16)
    w = w_ref[...]
    c = w.shape[1] // 2
    w1_16 = w[:, 0:c].astype(jnp.bfloat16)
    wc_16 = (w[:, c:2 * c] - w[:, 0:c]).astype(jnp.bfloat16)
    z1_ref[0] = lax.dot_general(fea16, w1_16, (((1,), (1,)), ((), ())),
                                preferred_element_type=F32)
    zc_ref[0] = lax.dot_general(fea16, wc_16, (((1,), (1,)), ((), ())),
                                preferred_element_type=F32)


def _knn_mix_call(x3, x3s, xsel, xall, feaT, w, interpret=False):
    b, _, _, n = x3.shape
    m = x3s.shape[-1]
    co = w.shape[0]
    return pl.pallas_call(
        _knn_mix_body,
        grid=(b,),
        in_specs=[
            pl.BlockSpec((1, 16, 8, n), lambda i: (i, 0, 0, 0)),
            pl.BlockSpec((1, 16, 8, m), lambda i: (i, 0, 0, 0)),
            pl.BlockSpec((1, m, 128), lambda i: (i, 0, 0)),
            pl.BlockSpec((1, n, 128), lambda i: (i, 0, 0)),
            pl.BlockSpec((1, n, 256), lambda i: (i, 0, 0)),
            pl.BlockSpec((co, w.shape[1]), lambda i: (0, 0)),
        ],
        out_specs=[
            pl.BlockSpec((1, m, SP_NEAR_K), lambda i: (i, 0, 0)),
            pl.BlockSpec((1, n, co), lambda i: (i, 0, 0)),
            pl.BlockSpec((1, n, co), lambda i: (i, 0, 0)),
        ],
        out_shape=[
            jax.ShapeDtypeStruct((b, m, SP_NEAR_K), I32),
            jax.ShapeDtypeStruct((b, n, co), F32),
            jax.ShapeDtypeStruct((b, n, co), F32),
        ],
        interpret=interpret,
    )(x3, x3s, xsel, xall, feaT, w)


def _pool_stats_body(g_ref, c_ref, bconv_ref, part_ref, mx_ref, mn_ref):
    g = g_ref[0]
    cb = c_ref[0] + bconv_ref[...]
    y0 = g[0] + cb
    mx = y0
    mn = y0
    s = y0
    ss = y0 * y0
    for k in range(1, SP_NEAR_K):
        yk = g[k] + cb
        mx = jnp.maximum(mx, yk)
        mn = jnp.minimum(mn, yk)
        s = s + yk
        ss = ss + yk * yk
    mx_ref[0] = mx
    mn_ref[0] = mn
    part_ref[0] = jnp.concatenate(
        [jnp.sum(s, axis=0, keepdims=True),
         jnp.sum(ss, axis=0, keepdims=True)], axis=0)


def _pool_stats_call(g4, c4, bconv2, interpret=False):
    b, _, m, co = g4.shape
    return pl.pallas_call(
        _pool_stats_body,
        grid=(b,),
        in_specs=[
            pl.BlockSpec((1, SP_NEAR_K, m, co), lambda i: (i, 0, 0, 0)),
            pl.BlockSpec((1, m, co), lambda i: (i, 0, 0)),
            pl.BlockSpec((1, co), lambda i: (0, 0)),
        ],
        out_specs=[
            pl.BlockSpec((1, 2, co), lambda i: (i, 0, 0)),
            pl.BlockSpec((1, m, co), lambda i: (i, 0, 0)),
            pl.BlockSpec((1, m, co), lambda i: (i, 0, 0)),
        ],
        out_shape=[
            jax.ShapeDtypeStruct((b, 2, co), F32),
            jax.ShapeDtypeStruct((b, m, co), F32),
            jax.ShapeDtypeStruct((b, m, co), F32),
        ],
        interpret=interpret,
    )(g4, c4, bconv2)


def _finalize_body(part_ref, mx_ref, mn_ref, gamma_ref, beta_ref, out_ref,
                   *, count):
    p = part_ref[...]
    s2 = jnp.sum(p[:, 0, :], axis=0, keepdims=True)
    ss2 = jnp.sum(p[:, 1, :], axis=0, keepdims=True)
    mu = s2 / count
    var = ss2 / count - mu * mu
    denom = jnp.sqrt(var + 1e-5)
    gamma = gamma_ref[...]
    beta = beta_ref[...]
    chosen = jnp.where(gamma >= 0.0, mx_ref[0], mn_ref[0])
    t = (chosen - mu) / denom * gamma + beta
    out_ref[0] = jnp.transpose(jax.nn.relu(t))


def _finalize_call(part, mx, mn, gamma2, beta2, interpret=False):
    b, m, co = mx.shape
    count = float(b * m * SP_NEAR_K)
    return pl.pallas_call(
        functools.partial(_finalize_body, count=count),
        grid=(b,),
        in_specs=[
            pl.BlockSpec((b, 2, co), lambda i: (0, 0, 0)),
            pl.BlockSpec((1, m, co), lambda i: (i, 0, 0)),
            pl.BlockSpec((1, m, co), lambda i: (i, 0, 0)),
            pl.BlockSpec((1, co), lambda i: (0, 0)),
            pl.BlockSpec((1, co), lambda i: (0, 0)),
        ],
        out_specs=pl.BlockSpec((1, co, m), lambda i: (i, 0, 0)),
        out_shape=jax.ShapeDtypeStruct((b, co, m), F32),
        interpret=interpret,
    )(part, mx, mn, gamma2, beta2)


def kernel(sparse_fea, stk_coor, n_stk_center, W, bconv, gamma, beta):
    b, n, cc = stk_coor.shape
    co = W.shape[0]
    m = M_CENTER

    x3 = stk_coor.reshape(b, n, 16, 8).transpose(0, 2, 3, 1)
    x3b = stk_coor.reshape(b, n, 16, 8).transpose(2, 0, 3, 1).reshape(16, 8 * b, n)

    fps_idx = _fps_call(x3b, stk_coor.reshape(b * n, cc)).reshape(b, m)
    fps_idx = fps_idx + (jnp.asarray(n_stk_center, I32) - jnp.asarray(m, I32))

    boff = (jnp.arange(b, dtype=I32) * n)[:, None]
    fps_flat = (fps_idx + boff).reshape(b * m)

    stk_sampled = _sc_gather(stk_coor.reshape(b * n, cc), fps_flat)
    stk_sampled = stk_sampled.reshape(b, m, cc)

    x3s = stk_sampled.reshape(b, m, 16, 8).transpose(0, 2, 3, 1)
    feaT = sparse_fea.transpose(0, 2, 1)

    knn_idx, z1t, zct = _knn_mix_call(x3, x3s, stk_sampled, stk_coor, feaT, W)

    knn_km = knn_idx.transpose(0, 2, 1)
    knn_flat = (knn_km + boff[:, :, None]).reshape(b * SP_NEAR_K * m)

    g = _sc_gather(z1t.reshape(b * n, co), knn_flat)
    c = _sc_gather(zct.reshape(b * n, co), fps_flat)
    g4 = g.reshape(b, SP_NEAR_K, m, co)
    c4 = c.reshape(b, m, co)

    part, mx, mn = _pool_stats_call(g4, c4, bconv.reshape(1, co))
    out_fea = _finalize_call(part, mx, mn, gamma.reshape(1, co),
                             beta.reshape(1, co))
    return (out_fea, stk_sampled)
